# baseline (speedup 1.0000x reference)
import jax
import jax.numpy as jnp
from jax import lax
from jax.experimental import pallas as pl
from jax.experimental.pallas import tpu as pltpu
from jax.experimental.pallas import tpu_sc as plsc

_NC = 2
_NS = 16
_HF = 64
_DW = 16
_R = 2000
_ZR = 128
_NP = 10240


def _leaky(x):
    return jnp.where(x >= 0.0, x, 0.2 * x)


def _l2n(x):
    n = jnp.sqrt(jnp.sum(x * x, axis=-1, keepdims=True))
    return x / jnp.maximum(n, 1e-12)


def _logmap0(b, sc):
    n = jnp.maximum(jnp.sqrt(jnp.sum(b * b, axis=-1, keepdims=True)), 1e-10)
    z = sc * n
    atanh = 0.5 * jnp.log((1.0 + z) / (1.0 - z))
    return (2.0 / sc) * atanh * b / n


def _expmap0(v, sc):
    n = jnp.maximum(jnp.sqrt(jnp.sum(v * v, axis=-1, keepdims=True)), 1e-10)
    return jnp.tanh(sc * n * 0.5) * v / (sc * n)


def _fuse_pre(e, b, s, ewt, eb1, bwt, bb1, swt, sb1, sc):
    he = jnp.dot(e, ewt, preferred_element_type=jnp.float32) + eb1
    hb = jnp.dot(_logmap0(b, sc), bwt, preferred_element_type=jnp.float32) + bb1
    hs = _l2n(jnp.dot(_l2n(s), swt, preferred_element_type=jnp.float32) + sb1)
    return he, jnp.concatenate([hb, hs], axis=1)


def _agg_split(plo_ref, phi_ref, d_ref, sc):
    deg = jnp.maximum(d_ref[...][:, :1], 1.0)
    e1 = _leaky(plo_ref[...] / deg)
    phi = phi_ref[...] / deg
    b1 = _expmap0(phi[:, :32], sc)
    s1 = _l2n(phi[:, 32:])
    return e1, b1, s1


def _wspecs(ed, bd, sd):
    return [
        pl.BlockSpec((ed, ed), lambda i: (0, 0)),
        pl.BlockSpec((1, ed), lambda i: (0, 0)),
        pl.BlockSpec((bd, bd), lambda i: (0, 0)),
        pl.BlockSpec((1, bd), lambda i: (0, 0)),
        pl.BlockSpec((sd, sd), lambda i: (0, 0)),
        pl.BlockSpec((1, sd), lambda i: (0, 0)),
    ]


def _tc_pre(c2, e_emb, b_emb, s_emb, w):
    n, ed = e_emb.shape
    bd = b_emb.shape[1]
    sd = s_emb.shape[1]

    def body(c_ref, e_ref, b_ref, s_ref, ewt, eb1, bwt, bb1, swt, sb1,
             hlo_ref, hhi_ref):
        sc = jnp.sqrt(c_ref[0, 0])
        hlo, hhi = _fuse_pre(e_ref[...], b_ref[...], s_ref[...],
                             ewt[...], eb1[...], bwt[...], bb1[...],
                             swt[...], sb1[...], sc)
        hlo_ref[...] = hlo
        hhi_ref[...] = hhi

    return pl.pallas_call(
        body,
        grid=(n // _R,),
        in_specs=[
            pl.BlockSpec(memory_space=pltpu.SMEM),
            pl.BlockSpec((_R, ed), lambda i: (i, 0)),
            pl.BlockSpec((_R, bd), lambda i: (i, 0)),
            pl.BlockSpec((_R, sd), lambda i: (i, 0)),
        ] + _wspecs(ed, bd, sd),
        out_specs=[
            pl.BlockSpec((_R, _HF), lambda i: (i, 0)),
            pl.BlockSpec((_R, _HF), lambda i: (i, 0)),
        ],
        out_shape=[
            jax.ShapeDtypeStruct((_NP, _HF), jnp.float32),
            jax.ShapeDtypeStruct((_NP, _HF), jnp.float32),
        ],
    )(c2, e_emb, b_emb, s_emb, *w)


def _tc_mid(c2, plo, phi, d, w, n, ed, bd, sd):
    def body(c_ref, plo_ref, phi_ref, d_ref, ewt, eb1, bwt, bb1, swt, sb1,
             hlo_ref, hhi_ref):
        sc = jnp.sqrt(c_ref[0, 0])
        e1, b1, s1 = _agg_split(plo_ref, phi_ref, d_ref, sc)
        hlo, hhi = _fuse_pre(e1, b1, s1, ewt[...], eb1[...], bwt[...],
                             bb1[...], swt[...], sb1[...], sc)
        hlo_ref[...] = hlo
        hhi_ref[...] = hhi

    return pl.pallas_call(
        body,
        grid=(n // _R,),
        in_specs=[
            pl.BlockSpec(memory_space=pltpu.SMEM),
            pl.BlockSpec((_R, _HF), lambda i: (i, 0)),
            pl.BlockSpec((_R, _HF), lambda i: (i, 0)),
            pl.BlockSpec((_R, _DW), lambda i: (i, 0)),
        ] + _wspecs(ed, bd, sd),
        out_specs=[
            pl.BlockSpec((_R, _HF), lambda i: (i, 0)),
            pl.BlockSpec((_R, _HF), lambda i: (i, 0)),
        ],
        out_shape=[
            jax.ShapeDtypeStruct((_NP, _HF), jnp.float32),
            jax.ShapeDtypeStruct((_NP, _HF), jnp.float32),
        ],
    )(c2, plo, phi, d, *w)


def _tc_post(c2, plo, phi, d, n, ed, bd, sd):
    def body(c_ref, plo_ref, phi_ref, d_ref, e_ref, b_ref, s_ref):
        sc = jnp.sqrt(c_ref[0, 0])
        e1, b1, s1 = _agg_split(plo_ref, phi_ref, d_ref, sc)
        e_ref[...] = e1
        b_ref[...] = b1
        s_ref[...] = s1

    return pl.pallas_call(
        body,
        grid=(n // _R,),
        in_specs=[
            pl.BlockSpec(memory_space=pltpu.SMEM),
            pl.BlockSpec((_R, _HF), lambda i: (i, 0)),
            pl.BlockSpec((_R, _HF), lambda i: (i, 0)),
            pl.BlockSpec((_R, _DW), lambda i: (i, 0)),
        ],
        out_specs=[
            pl.BlockSpec((_R, ed), lambda i: (i, 0)),
            pl.BlockSpec((_R, bd), lambda i: (i, 0)),
            pl.BlockSpec((_R, sd), lambda i: (i, 0)),
        ],
        out_shape=[
            jax.ShapeDtypeStruct((n, ed), jnp.float32),
            jax.ShapeDtypeStruct((n, bd), jnp.float32),
            jax.ShapeDtypeStruct((n, sd), jnp.float32),
        ],
    )(c2, plo, phi, d)


def _sc_agg(hlo, hhi, src3, dst3, z64, z16, ones_in, with_deg):
    n = hlo.shape[0]
    nch, cb = src3.shape[1], src3.shape[2]
    rpt = n // _NS

    mesh = plsc.VectorSubcoreMesh(core_axis_name="c", subcore_axis_name="s",
                                  num_cores=_NC, num_subcores=_NS)
    out_type = [jax.ShapeDtypeStruct((n, _HF), jnp.float32),
                jax.ShapeDtypeStruct((n, _HF), jnp.float32)]
    scratch = [
        pltpu.VMEM((nch, cb), jnp.int32),
        pltpu.VMEM((nch, cb), jnp.int32),
        pltpu.VMEM((4, cb, _HF), jnp.float32),
        pltpu.VMEM_SHARED((n, _HF), jnp.float32),
        pltpu.SemaphoreType.DMA,
        pltpu.SemaphoreType.DMA,
        pltpu.SemaphoreType.DMA,
        pltpu.SemaphoreType.DMA,
        pltpu.SemaphoreType.DMA,
        pltpu.SemaphoreType.DMA,
        pltpu.SemaphoreType.DMA,
        pltpu.SemaphoreType.DMA,
    ]
    if with_deg:
        out_type.append(jax.ShapeDtypeStruct((n, _DW), jnp.float32))
        scratch += [
            pltpu.VMEM((cb, _DW), jnp.float32),
            pltpu.VMEM_SHARED((n, _DW), jnp.float32),
        ]

    def body(hlo_hbm, hhi_hbm, src_hbm, dst_hbm, z64_hbm, *rest):
        if with_deg:
            z16_hbm, ones_hbm, plo_out, phi_out, d_out, srcv, dstv, rows, \
                acc, gs0, gs1, gs2, gs3, ss0, ss1, ss2, ss3, ones, dacc = rest
        else:
            plo_out, phi_out, srcv, dstv, rows, acc, \
                gs0, gs1, gs2, gs3, ss0, ss1, ss2, ss3 = rest
        gsems = (gs0, gs1, gs2, gs3)
        ssems = (ss0, ss1, ss2, ss3)
        cid = lax.axis_index("c")
        sid = lax.axis_index("s")
        base = sid * rpt

        pltpu.sync_copy(z64_hbm, acc.at[pl.ds(base, rpt)])
        if with_deg:
            pltpu.sync_copy(z16_hbm, dacc.at[pl.ds(base, rpt)])
            pltpu.sync_copy(ones_hbm, ones)
        pltpu.sync_copy(src_hbm.at[sid], srcv)
        pltpu.sync_copy(dst_hbm.at[sid], dstv)
        plsc.subcore_barrier()

        def _gather(j, b, sem):
            @pl.when(cid == 0)
            def _():
                pltpu.async_copy(hlo_hbm.at[srcv.at[j]], rows.at[b], sem)

            @pl.when(cid == 1)
            def _():
                pltpu.async_copy(hhi_hbm.at[srcv.at[j]], rows.at[b], sem)

        def _gwait(j, b, sem):
            pltpu.make_async_copy(hlo_hbm.at[srcv.at[j]], rows.at[b], sem).wait()

        def _scat(j, b):
            pltpu.async_copy(rows.at[b], acc.at[dstv.at[j]], ssems[b], add=True)
            if with_deg:
                @pl.when(cid == 1)
                def _():
                    pltpu.sync_copy(ones, dacc.at[dstv.at[j]], add=True)

        def _swait(j, b):
            pltpu.make_async_copy(rows.at[b], acc.at[dstv.at[j]],
                                  ssems[b]).wait()

        nloop = nch // 4
        for b in range(3):
            _gather(b, b, gsems[b])

        def _quad(g, carry):
            for b in range(4):
                j = 4 * g + b
                bp = (b + 3) % 4

                _gwait(j, b, gsems[b])
                _scat(j, b)

                @pl.when(j >= 1)
                def _():
                    _swait(j - 1, bp)

                @pl.when(j + 3 < nch)
                def _():
                    _gather(j + 3, bp, gsems[bp])
            return carry
        lax.fori_loop(0, nloop, _quad, 0)
        _swait(nch - 1, (nch - 1) % 4)
        plsc.subcore_barrier()

        sl = pl.ds(base, rpt)

        @pl.when(cid == 0)
        def _():
            pltpu.sync_copy(acc.at[sl], plo_out.at[sl])

        @pl.when(cid == 1)
        def _():
            pltpu.sync_copy(acc.at[sl], phi_out.at[sl])
        if with_deg:
            @pl.when(cid == 1)
            def _():
                pltpu.sync_copy(dacc.at[sl], d_out.at[sl])

    f = pl.kernel(body, out_type=out_type, mesh=mesh, scratch_types=scratch,
                  compiler_params=pltpu.CompilerParams(use_tc_tiling_on_sc=False))
    if with_deg:
        return f(hlo, hhi, src3, dst3, z64, z16, ones_in)
    return f(hlo, hhi, src3, dst3, z64)


def kernel(e_emb, b_emb, s_emb, edge_index, b_curvature, s_curvature,
           eW, eb, bW, bb, sW, sb):
    n, ed = e_emb.shape
    bd = b_emb.shape[1]
    sd = s_emb.shape[1]
    e_edges = edge_index.shape[1]
    per_t = e_edges // _NS
    cb = 125
    nch = per_t // cb
    src3 = edge_index[0].reshape(_NS, nch, cb)
    dst3 = edge_index[1].reshape(_NS, nch, cb)
    c2 = b_curvature.reshape(1, 1)

    def wlayer(l):
        return (eW[l].T, eb[l].reshape(1, ed), bW[l].T, bb[l].reshape(1, bd),
                sW[l].T, sb[l].reshape(1, sd))

    rpt = _NP // _NS
    z64 = jnp.zeros((rpt, _HF), jnp.float32)
    z16 = jnp.zeros((rpt, _DW), jnp.float32)
    ones_in = jnp.ones((cb, _DW), jnp.float32)

    hlo0, hhi0 = _tc_pre(c2, e_emb, b_emb, s_emb, wlayer(0))
    plo0, phi0, d0 = _sc_agg(hlo0, hhi0, src3, dst3, z64, z16, ones_in,
                             with_deg=True)
    hlo1, hhi1 = _tc_mid(c2, plo0, phi0, d0, wlayer(1), n, ed, bd, sd)
    plo1, phi1 = _sc_agg(hlo1, hhi1, src3, dst3, z64, z16, ones_in,
                         with_deg=False)
    e2, b2, s2 = _tc_post(c2, plo1, phi1, d0, n, ed, bd, sd)
    return (e2, b2, s2)


# speedup vs baseline: 1.0682x; 1.0682x over previous
import jax
import jax.numpy as jnp
from jax import lax
from jax.experimental import pallas as pl
from jax.experimental.pallas import tpu as pltpu
from jax.experimental.pallas import tpu_sc as plsc

_NC = 2
_NS = 16
_HF = 64
_DW = 16
_R = 2000
_ZR = 128
_NP = 10240


def _leaky(x):
    return jnp.where(x >= 0.0, x, 0.2 * x)


def _l2n(x):
    n = jnp.sqrt(jnp.sum(x * x, axis=-1, keepdims=True))
    return x / jnp.maximum(n, 1e-12)


def _logmap0(b, sc):
    n = jnp.maximum(jnp.sqrt(jnp.sum(b * b, axis=-1, keepdims=True)), 1e-10)
    z = sc * n
    atanh = 0.5 * jnp.log((1.0 + z) / (1.0 - z))
    return (2.0 / sc) * atanh * b / n


def _expmap0(v, sc):
    n = jnp.maximum(jnp.sqrt(jnp.sum(v * v, axis=-1, keepdims=True)), 1e-10)
    return jnp.tanh(sc * n * 0.5) * v / (sc * n)


def _fuse_pre(e, b, s, ewt, eb1, bwt, bb1, swt, sb1, sc):
    he = jnp.dot(e, ewt, preferred_element_type=jnp.float32) + eb1
    hb = jnp.dot(_logmap0(b, sc), bwt, preferred_element_type=jnp.float32) + bb1
    hs = _l2n(jnp.dot(_l2n(s), swt, preferred_element_type=jnp.float32) + sb1)
    return jnp.concatenate([he, hb, hs], axis=1)


def _agg_split(plo_ref, phi_ref, d_ref, sc):
    deg = jnp.maximum(d_ref[...][:, :1], 1.0)
    e1 = _leaky(plo_ref[...] / deg)
    phi = phi_ref[...] / deg
    b1 = _expmap0(phi[:, :32], sc)
    s1 = _l2n(phi[:, 32:])
    return e1, b1, s1


def _wspecs(ed, bd, sd):
    return [
        pl.BlockSpec((ed, ed), lambda i: (0, 0)),
        pl.BlockSpec((1, ed), lambda i: (0, 0)),
        pl.BlockSpec((bd, bd), lambda i: (0, 0)),
        pl.BlockSpec((1, bd), lambda i: (0, 0)),
        pl.BlockSpec((sd, sd), lambda i: (0, 0)),
        pl.BlockSpec((1, sd), lambda i: (0, 0)),
    ]


def _tc_pre(c2, e_emb, b_emb, s_emb, w):
    n, ed = e_emb.shape
    bd = b_emb.shape[1]
    sd = s_emb.shape[1]

    def body(c_ref, e_ref, b_ref, s_ref, ewt, eb1, bwt, bb1, swt, sb1, h_ref):
        sc = jnp.sqrt(c_ref[0, 0])
        h_ref[...] = _fuse_pre(e_ref[...], b_ref[...], s_ref[...],
                               ewt[...], eb1[...], bwt[...], bb1[...],
                               swt[...], sb1[...], sc)

    return pl.pallas_call(
        body,
        grid=(n // _R,),
        in_specs=[
            pl.BlockSpec(memory_space=pltpu.SMEM),
            pl.BlockSpec((_R, ed), lambda i: (i, 0)),
            pl.BlockSpec((_R, bd), lambda i: (i, 0)),
            pl.BlockSpec((_R, sd), lambda i: (i, 0)),
        ] + _wspecs(ed, bd, sd),
        out_specs=pl.BlockSpec((_R, 2 * _HF), lambda i: (i, 0)),
        out_shape=jax.ShapeDtypeStruct((_NP, 2 * _HF), jnp.float32),
    )(c2, e_emb, b_emb, s_emb, *w)


def _tc_mid(c2, plo, phi, d, w, n, ed, bd, sd):
    def body(c_ref, plo_ref, phi_ref, d_ref, ewt, eb1, bwt, bb1, swt, sb1,
             h_ref):
        sc = jnp.sqrt(c_ref[0, 0])
        e1, b1, s1 = _agg_split(plo_ref, phi_ref, d_ref, sc)
        h_ref[...] = _fuse_pre(e1, b1, s1, ewt[...], eb1[...], bwt[...],
                               bb1[...], swt[...], sb1[...], sc)

    return pl.pallas_call(
        body,
        grid=(n // _R,),
        in_specs=[
            pl.BlockSpec(memory_space=pltpu.SMEM),
            pl.BlockSpec((_R, _HF), lambda i: (i, 0)),
            pl.BlockSpec((_R, _HF), lambda i: (i, 0)),
            pl.BlockSpec((_R, _DW), lambda i: (i, 0)),
        ] + _wspecs(ed, bd, sd),
        out_specs=pl.BlockSpec((_R, 2 * _HF), lambda i: (i, 0)),
        out_shape=jax.ShapeDtypeStruct((_NP, 2 * _HF), jnp.float32),
    )(c2, plo, phi, d, *w)


def _tc_post(c2, plo, phi, d, n, ed, bd, sd):
    def body(c_ref, plo_ref, phi_ref, d_ref, e_ref, b_ref, s_ref):
        sc = jnp.sqrt(c_ref[0, 0])
        e1, b1, s1 = _agg_split(plo_ref, phi_ref, d_ref, sc)
        e_ref[...] = e1
        b_ref[...] = b1
        s_ref[...] = s1

    return pl.pallas_call(
        body,
        grid=(n // _R,),
        in_specs=[
            pl.BlockSpec(memory_space=pltpu.SMEM),
            pl.BlockSpec((_R, _HF), lambda i: (i, 0)),
            pl.BlockSpec((_R, _HF), lambda i: (i, 0)),
            pl.BlockSpec((_R, _DW), lambda i: (i, 0)),
        ],
        out_specs=[
            pl.BlockSpec((_R, ed), lambda i: (i, 0)),
            pl.BlockSpec((_R, bd), lambda i: (i, 0)),
            pl.BlockSpec((_R, sd), lambda i: (i, 0)),
        ],
        out_shape=[
            jax.ShapeDtypeStruct((n, ed), jnp.float32),
            jax.ShapeDtypeStruct((n, bd), jnp.float32),
            jax.ShapeDtypeStruct((n, sd), jnp.float32),
        ],
    )(c2, plo, phi, d)


def _sc_agg(hview, srce3, srco3, dst3, z64, z16, ones_in, with_deg):
    n = hview.shape[0] // 2
    nch, cb = dst3.shape[1], dst3.shape[2]
    rpt = n // _NS

    mesh = plsc.VectorSubcoreMesh(core_axis_name="c", subcore_axis_name="s",
                                  num_cores=_NC, num_subcores=_NS)
    out_type = [jax.ShapeDtypeStruct((n, _HF), jnp.float32),
                jax.ShapeDtypeStruct((n, _HF), jnp.float32)]
    scratch = [
        pltpu.VMEM((nch, cb), jnp.int32),
        pltpu.VMEM((nch, cb), jnp.int32),
        pltpu.VMEM((4, cb, _HF), jnp.float32),
        pltpu.VMEM_SHARED((n, _HF), jnp.float32),
        pltpu.SemaphoreType.DMA,
        pltpu.SemaphoreType.DMA,
        pltpu.SemaphoreType.DMA,
        pltpu.SemaphoreType.DMA,
        pltpu.SemaphoreType.DMA,
        pltpu.SemaphoreType.DMA,
        pltpu.SemaphoreType.DMA,
        pltpu.SemaphoreType.DMA,
    ]
    if with_deg:
        out_type.append(jax.ShapeDtypeStruct((n, _DW), jnp.float32))
        scratch += [
            pltpu.VMEM((cb, _DW), jnp.float32),
            pltpu.VMEM_SHARED((n, _DW), jnp.float32),
        ]

    def body(hview_hbm, srce_hbm, srco_hbm, dst_hbm, z64_hbm, *rest):
        if with_deg:
            z16_hbm, ones_hbm, plo_out, phi_out, d_out, srcv, dstv, rows, \
                acc, gs0, gs1, gs2, gs3, ss0, ss1, ss2, ss3, ones, dacc = rest
        else:
            plo_out, phi_out, srcv, dstv, rows, acc, \
                gs0, gs1, gs2, gs3, ss0, ss1, ss2, ss3 = rest
        gsems = (gs0, gs1, gs2, gs3)
        ssems = (ss0, ss1, ss2, ss3)
        cid = lax.axis_index("c")
        sid = lax.axis_index("s")
        base = sid * rpt

        pltpu.sync_copy(z64_hbm, acc.at[pl.ds(base, rpt)])
        if with_deg:
            pltpu.sync_copy(z16_hbm, dacc.at[pl.ds(base, rpt)])
            pltpu.sync_copy(ones_hbm, ones)
        @pl.when(cid == 0)
        def _():
            pltpu.sync_copy(srce_hbm.at[sid], srcv)

        @pl.when(cid == 1)
        def _():
            pltpu.sync_copy(srco_hbm.at[sid], srcv)
        pltpu.sync_copy(dst_hbm.at[sid], dstv)
        plsc.subcore_barrier()

        def _gather(j, b, sem):
            pltpu.async_copy(hview_hbm.at[srcv.at[j]], rows.at[b], sem)

        def _gwait(j, b, sem):
            pltpu.make_async_copy(hview_hbm.at[srcv.at[j]], rows.at[b],
                                  sem).wait()

        def _scat(j, b):
            pltpu.sync_copy(rows.at[b], acc.at[dstv.at[j]], add=True)
            if with_deg:
                @pl.when(cid == 1)
                def _():
                    pltpu.sync_copy(ones, dacc.at[dstv.at[j]], add=True)

        nloop = nch // 4
        for b in range(3):
            _gather(b, b, gsems[b])

        def _quad(g, carry):
            for b in range(4):
                j = 4 * g + b
                bp = (b + 3) % 4

                @pl.when(j + 3 < nch)
                def _():
                    _gather(j + 3, bp, gsems[bp])
                _gwait(j, b, gsems[b])
                _scat(j, b)
            return carry
        lax.fori_loop(0, nloop, _quad, 0)
        plsc.subcore_barrier()

        sl = pl.ds(base, rpt)

        @pl.when(cid == 0)
        def _():
            pltpu.sync_copy(acc.at[sl], plo_out.at[sl])

        @pl.when(cid == 1)
        def _():
            pltpu.sync_copy(acc.at[sl], phi_out.at[sl])
        if with_deg:
            @pl.when(cid == 1)
            def _():
                pltpu.sync_copy(dacc.at[sl], d_out.at[sl])

    f = pl.kernel(body, out_type=out_type, mesh=mesh, scratch_types=scratch,
                  compiler_params=pltpu.CompilerParams(use_tc_tiling_on_sc=False))
    if with_deg:
        return f(hview, srce3, srco3, dst3, z64, z16, ones_in)
    return f(hview, srce3, srco3, dst3, z64)


def kernel(e_emb, b_emb, s_emb, edge_index, b_curvature, s_curvature,
           eW, eb, bW, bb, sW, sb):
    n, ed = e_emb.shape
    bd = b_emb.shape[1]
    sd = s_emb.shape[1]
    e_edges = edge_index.shape[1]
    per_t = e_edges // _NS
    cb = 125
    nch = per_t // cb
    src = edge_index[0]
    srce3 = (src * 2).reshape(_NS, nch, cb)
    srco3 = (src * 2 + 1).reshape(_NS, nch, cb)
    dst3 = edge_index[1].reshape(_NS, nch, cb)
    c2 = b_curvature.reshape(1, 1)

    def wlayer(l):
        return (eW[l].T, eb[l].reshape(1, ed), bW[l].T, bb[l].reshape(1, bd),
                sW[l].T, sb[l].reshape(1, sd))

    rpt = _NP // _NS
    z64 = jnp.zeros((rpt, _HF), jnp.float32)
    z16 = jnp.zeros((rpt, _DW), jnp.float32)
    ones_in = jnp.ones((cb, _DW), jnp.float32)

    h0 = _tc_pre(c2, e_emb, b_emb, s_emb, wlayer(0))
    plo0, phi0, d0 = _sc_agg(h0.reshape(2 * _NP, _HF), srce3, srco3, dst3,
                             z64, z16, ones_in, with_deg=True)
    h1 = _tc_mid(c2, plo0, phi0, d0, wlayer(1), n, ed, bd, sd)
    plo1, phi1 = _sc_agg(h1.reshape(2 * _NP, _HF), srce3, srco3, dst3,
                         z64, z16, ones_in, with_deg=False)
    e2, b2, s2 = _tc_post(c2, plo1, phi1, d0, n, ed, bd, sd)
    return (e2, b2, s2)


# speedup vs baseline: 1.0791x; 1.0102x over previous
import jax
import jax.numpy as jnp
from jax import lax
from jax.experimental import pallas as pl
from jax.experimental.pallas import tpu as pltpu
from jax.experimental.pallas import tpu_sc as plsc

_NC = 2
_NS = 16
_HF = 64
_DW = 16
_R = 2000
_ZR = 128
_NP = 10240


def _leaky(x):
    return jnp.where(x >= 0.0, x, 0.2 * x)


def _l2n(x):
    n = jnp.sqrt(jnp.sum(x * x, axis=-1, keepdims=True))
    return x / jnp.maximum(n, 1e-12)


def _logmap0(b, sc):
    n = jnp.maximum(jnp.sqrt(jnp.sum(b * b, axis=-1, keepdims=True)), 1e-10)
    z = sc * n
    atanh = 0.5 * jnp.log((1.0 + z) / (1.0 - z))
    return (2.0 / sc) * atanh * b / n


def _expmap0(v, sc):
    n = jnp.maximum(jnp.sqrt(jnp.sum(v * v, axis=-1, keepdims=True)), 1e-10)
    return jnp.tanh(sc * n * 0.5) * v / (sc * n)


def _fuse_pre(e, b, s, ewt, eb1, bwt, bb1, swt, sb1, sc):
    he = jnp.dot(e, ewt, preferred_element_type=jnp.float32) + eb1
    hb = jnp.dot(_logmap0(b, sc), bwt, preferred_element_type=jnp.float32) + bb1
    hs = _l2n(jnp.dot(_l2n(s), swt, preferred_element_type=jnp.float32) + sb1)
    return jnp.concatenate([he, hb, hs], axis=1)


def _agg_split(plo_ref, phi_ref, d_ref, sc):
    deg = jnp.maximum(d_ref[...][:, :1], 1.0)
    e1 = _leaky(plo_ref[...] / deg)
    phi = phi_ref[...] / deg
    b1 = _expmap0(phi[:, :32], sc)
    s1 = _l2n(phi[:, 32:])
    return e1, b1, s1


def _wspecs(ed, bd, sd):
    return [
        pl.BlockSpec((ed, ed), lambda i: (0, 0)),
        pl.BlockSpec((1, ed), lambda i: (0, 0)),
        pl.BlockSpec((bd, bd), lambda i: (0, 0)),
        pl.BlockSpec((1, bd), lambda i: (0, 0)),
        pl.BlockSpec((sd, sd), lambda i: (0, 0)),
        pl.BlockSpec((1, sd), lambda i: (0, 0)),
    ]


def _tc_pre(c2, e_emb, b_emb, s_emb, w):
    n, ed = e_emb.shape
    bd = b_emb.shape[1]
    sd = s_emb.shape[1]

    def body(c_ref, e_ref, b_ref, s_ref, ewt, eb1, bwt, bb1, swt, sb1, h_ref):
        sc = jnp.sqrt(c_ref[0, 0])
        h_ref[...] = _fuse_pre(e_ref[...], b_ref[...], s_ref[...],
                               ewt[...], eb1[...], bwt[...], bb1[...],
                               swt[...], sb1[...], sc)

    return pl.pallas_call(
        body,
        grid=(n // _R,),
        in_specs=[
            pl.BlockSpec(memory_space=pltpu.SMEM),
            pl.BlockSpec((_R, ed), lambda i: (i, 0)),
            pl.BlockSpec((_R, bd), lambda i: (i, 0)),
            pl.BlockSpec((_R, sd), lambda i: (i, 0)),
        ] + _wspecs(ed, bd, sd),
        out_specs=pl.BlockSpec((_R, 2 * _HF), lambda i: (i, 0)),
        out_shape=jax.ShapeDtypeStruct((_NP, 2 * _HF), jnp.float32),
    )(c2, e_emb, b_emb, s_emb, *w)


def _tc_mid(c2, plo, phi, d, w, n, ed, bd, sd):
    def body(c_ref, plo_ref, phi_ref, d_ref, ewt, eb1, bwt, bb1, swt, sb1,
             h_ref):
        sc = jnp.sqrt(c_ref[0, 0])
        e1, b1, s1 = _agg_split(plo_ref, phi_ref, d_ref, sc)
        h_ref[...] = _fuse_pre(e1, b1, s1, ewt[...], eb1[...], bwt[...],
                               bb1[...], swt[...], sb1[...], sc)

    return pl.pallas_call(
        body,
        grid=(n // _R,),
        in_specs=[
            pl.BlockSpec(memory_space=pltpu.SMEM),
            pl.BlockSpec((_R, _HF), lambda i: (i, 0)),
            pl.BlockSpec((_R, _HF), lambda i: (i, 0)),
            pl.BlockSpec((_R, _DW), lambda i: (i, 0)),
        ] + _wspecs(ed, bd, sd),
        out_specs=pl.BlockSpec((_R, 2 * _HF), lambda i: (i, 0)),
        out_shape=jax.ShapeDtypeStruct((_NP, 2 * _HF), jnp.float32),
    )(c2, plo, phi, d, *w)


def _tc_post(c2, plo, phi, d, n, ed, bd, sd):
    def body(c_ref, plo_ref, phi_ref, d_ref, e_ref, b_ref, s_ref):
        sc = jnp.sqrt(c_ref[0, 0])
        e1, b1, s1 = _agg_split(plo_ref, phi_ref, d_ref, sc)
        e_ref[...] = e1
        b_ref[...] = b1
        s_ref[...] = s1

    return pl.pallas_call(
        body,
        grid=(n // _R,),
        in_specs=[
            pl.BlockSpec(memory_space=pltpu.SMEM),
            pl.BlockSpec((_R, _HF), lambda i: (i, 0)),
            pl.BlockSpec((_R, _HF), lambda i: (i, 0)),
            pl.BlockSpec((_R, _DW), lambda i: (i, 0)),
        ],
        out_specs=[
            pl.BlockSpec((_R, ed), lambda i: (i, 0)),
            pl.BlockSpec((_R, bd), lambda i: (i, 0)),
            pl.BlockSpec((_R, sd), lambda i: (i, 0)),
        ],
        out_shape=[
            jax.ShapeDtypeStruct((n, ed), jnp.float32),
            jax.ShapeDtypeStruct((n, bd), jnp.float32),
            jax.ShapeDtypeStruct((n, sd), jnp.float32),
        ],
    )(c2, plo, phi, d)


def _sc_agg(hview, srce3, srco3, dst3, z64, z16, ones_in, with_deg):
    n = hview.shape[0] // 2
    nch, cb = dst3.shape[1], dst3.shape[2]
    rpt = n // _NS

    mesh = plsc.VectorSubcoreMesh(core_axis_name="c", subcore_axis_name="s",
                                  num_cores=_NC, num_subcores=_NS)
    out_type = [jax.ShapeDtypeStruct((n, _HF), jnp.float32),
                jax.ShapeDtypeStruct((n, _HF), jnp.float32)]
    nr, ni = 5, 10
    scratch = [
        pltpu.VMEM((ni, cb), jnp.int32),
        pltpu.VMEM((ni, cb), jnp.int32),
        pltpu.VMEM((nr, cb, _HF), jnp.float32),
        pltpu.VMEM_SHARED((n, _HF), jnp.float32),
    ]
    scratch += [pltpu.SemaphoreType.DMA] * nr
    scratch += [pltpu.SemaphoreType.DMA] * ni
    if with_deg:
        out_type.append(jax.ShapeDtypeStruct((n, _DW), jnp.float32))
        scratch += [
            pltpu.VMEM((cb, _DW), jnp.float32),
            pltpu.VMEM_SHARED((n, _DW), jnp.float32),
        ]

    def body(hview_hbm, srce_hbm, srco_hbm, dst_hbm, z64_hbm, *rest):
        if with_deg:
            z16_hbm, ones_hbm = rest[:2]
            rest = rest[2:]
        plo_out, phi_out = rest[:2]
        rest = rest[2:]
        if with_deg:
            d_out = rest[0]
            rest = rest[1:]
        sidx, didx, rows, acc = rest[:4]
        gsems = rest[4:4 + nr]
        isems = rest[4 + nr:4 + nr + ni]
        if with_deg:
            ones, dacc = rest[4 + nr + ni:]
        cid = lax.axis_index("c")
        sid = lax.axis_index("s")
        base = sid * rpt

        pltpu.sync_copy(z64_hbm, acc.at[pl.ds(base, rpt)])
        if with_deg:
            pltpu.sync_copy(z16_hbm, dacc.at[pl.ds(base, rpt)])
            pltpu.sync_copy(ones_hbm, ones)
        plsc.subcore_barrier()

        def _idx_issue(j, m):
            @pl.when(cid == 0)
            def _():
                pltpu.async_copy(srce_hbm.at[sid, j], sidx.at[m], isems[m])

            @pl.when(cid == 1)
            def _():
                pltpu.async_copy(srco_hbm.at[sid, j], sidx.at[m], isems[m])
            pltpu.async_copy(dst_hbm.at[sid, j], didx.at[m], isems[m])

        def _idx_wait(j, m):
            pltpu.make_async_copy(dst_hbm.at[sid, j], sidx.at[m],
                                  isems[m]).wait()
            pltpu.make_async_copy(dst_hbm.at[sid, j], didx.at[m],
                                  isems[m]).wait()

        def _gather(j, b, m):
            pltpu.async_copy(hview_hbm.at[sidx.at[m]], rows.at[b], gsems[b])

        def _gwait(j, b, m):
            pltpu.make_async_copy(hview_hbm.at[sidx.at[m]], rows.at[b],
                                  gsems[b]).wait()

        def _scat(j, b, m):
            pltpu.sync_copy(rows.at[b], acc.at[didx.at[m]], add=True)
            if with_deg:
                @pl.when(cid == 1)
                def _():
                    pltpu.sync_copy(ones, dacc.at[didx.at[m]], add=True)

        for j in range(ni - 1):
            _idx_issue(j, j % ni)
        for j in range(nr - 1):
            _idx_wait(j, j % ni)
            _gather(j, j % nr, j % ni)

        def _slot(g, carry):
            for t in range(ni):
                j = ni * g + t
                b = t % nr
                _gwait(j, b, t)
                _scat(j, b, t)

                @pl.when(j + ni - 1 < nch)
                def _():
                    _idx_issue(j + ni - 1, (t + ni - 1) % ni)

                @pl.when(j + nr - 1 < nch)
                def _():
                    _idx_wait(j + nr - 1, (t + nr - 1) % ni)
                    _gather(j + nr - 1, (b + nr - 1) % nr, (t + nr - 1) % ni)
            return carry
        lax.fori_loop(0, nch // ni, _slot, 0)
        plsc.subcore_barrier()

        sl = pl.ds(base, rpt)

        @pl.when(cid == 0)
        def _():
            pltpu.sync_copy(acc.at[sl], plo_out.at[sl])

        @pl.when(cid == 1)
        def _():
            pltpu.sync_copy(acc.at[sl], phi_out.at[sl])
        if with_deg:
            @pl.when(cid == 1)
            def _():
                pltpu.sync_copy(dacc.at[sl], d_out.at[sl])

    f = pl.kernel(body, out_type=out_type, mesh=mesh, scratch_types=scratch,
                  compiler_params=pltpu.CompilerParams(use_tc_tiling_on_sc=False))
    if with_deg:
        return f(hview, srce3, srco3, dst3, z64, z16, ones_in)
    return f(hview, srce3, srco3, dst3, z64)


def kernel(e_emb, b_emb, s_emb, edge_index, b_curvature, s_curvature,
           eW, eb, bW, bb, sW, sb):
    n, ed = e_emb.shape
    bd = b_emb.shape[1]
    sd = s_emb.shape[1]
    e_edges = edge_index.shape[1]
    per_t = e_edges // _NS
    cb = 125
    nch = per_t // cb
    src = edge_index[0]
    srce3 = (src * 2).reshape(_NS, nch, cb)
    srco3 = (src * 2 + 1).reshape(_NS, nch, cb)
    dst3 = edge_index[1].reshape(_NS, nch, cb)
    c2 = b_curvature.reshape(1, 1)

    def wlayer(l):
        return (eW[l].T, eb[l].reshape(1, ed), bW[l].T, bb[l].reshape(1, bd),
                sW[l].T, sb[l].reshape(1, sd))

    rpt = _NP // _NS
    z64 = jnp.zeros((rpt, _HF), jnp.float32)
    z16 = jnp.zeros((rpt, _DW), jnp.float32)
    ones_in = jnp.ones((cb, _DW), jnp.float32)

    h0 = _tc_pre(c2, e_emb, b_emb, s_emb, wlayer(0))
    plo0, phi0, d0 = _sc_agg(h0.reshape(2 * _NP, _HF), srce3, srco3, dst3,
                             z64, z16, ones_in, with_deg=True)
    h1 = _tc_mid(c2, plo0, phi0, d0, wlayer(1), n, ed, bd, sd)
    plo1, phi1 = _sc_agg(h1.reshape(2 * _NP, _HF), srce3, srco3, dst3,
                         z64, z16, ones_in, with_deg=False)
    e2, b2, s2 = _tc_post(c2, plo1, phi1, d0, n, ed, bd, sd)
    return (e2, b2, s2)


# speedup vs baseline: 1.0817x; 1.0024x over previous
import jax
import jax.numpy as jnp
from jax import lax
from jax.experimental import pallas as pl
from jax.experimental.pallas import tpu as pltpu
from jax.experimental.pallas import tpu_sc as plsc

_NC = 2
_NS = 16
_HF = 64
_DW = 16
_R = 2000
_ZR = 128
_NP = 10240


def _leaky(x):
    return jnp.where(x >= 0.0, x, 0.2 * x)


def _l2n(x):
    n = jnp.sqrt(jnp.sum(x * x, axis=-1, keepdims=True))
    return x / jnp.maximum(n, 1e-12)


def _logmap0(b, sc):
    n = jnp.maximum(jnp.sqrt(jnp.sum(b * b, axis=-1, keepdims=True)), 1e-10)
    z = sc * n
    atanh = 0.5 * jnp.log((1.0 + z) / (1.0 - z))
    return (2.0 / sc) * atanh * b / n


def _expmap0(v, sc):
    n = jnp.maximum(jnp.sqrt(jnp.sum(v * v, axis=-1, keepdims=True)), 1e-10)
    return jnp.tanh(sc * n * 0.5) * v / (sc * n)


def _fuse_pre(e, b, s, ewt, eb1, bwt, bb1, swt, sb1, sc):
    he = jnp.dot(e, ewt, preferred_element_type=jnp.float32) + eb1
    hb = jnp.dot(_logmap0(b, sc), bwt, preferred_element_type=jnp.float32) + bb1
    hs = _l2n(jnp.dot(_l2n(s), swt, preferred_element_type=jnp.float32) + sb1)
    return jnp.concatenate([he, hb, hs], axis=1)


def _agg_split(plo_ref, phi_ref, da_ref, db_ref, sc):
    deg = jnp.maximum(da_ref[...][:, :1] + db_ref[...][:, :1], 1.0)
    e1 = _leaky(plo_ref[...] / deg)
    phi = phi_ref[...] / deg
    b1 = _expmap0(phi[:, :32], sc)
    s1 = _l2n(phi[:, 32:])
    return e1, b1, s1


def _wspecs(ed, bd, sd):
    return [
        pl.BlockSpec((ed, ed), lambda i: (0, 0)),
        pl.BlockSpec((1, ed), lambda i: (0, 0)),
        pl.BlockSpec((bd, bd), lambda i: (0, 0)),
        pl.BlockSpec((1, bd), lambda i: (0, 0)),
        pl.BlockSpec((sd, sd), lambda i: (0, 0)),
        pl.BlockSpec((1, sd), lambda i: (0, 0)),
    ]


def _tc_pre(c2, e_emb, b_emb, s_emb, w):
    n, ed = e_emb.shape
    bd = b_emb.shape[1]
    sd = s_emb.shape[1]

    def body(c_ref, e_ref, b_ref, s_ref, ewt, eb1, bwt, bb1, swt, sb1, h_ref):
        sc = jnp.sqrt(c_ref[0, 0])
        h_ref[...] = _fuse_pre(e_ref[...], b_ref[...], s_ref[...],
                               ewt[...], eb1[...], bwt[...], bb1[...],
                               swt[...], sb1[...], sc)

    return pl.pallas_call(
        body,
        grid=(n // _R,),
        in_specs=[
            pl.BlockSpec(memory_space=pltpu.SMEM),
            pl.BlockSpec((_R, ed), lambda i: (i, 0)),
            pl.BlockSpec((_R, bd), lambda i: (i, 0)),
            pl.BlockSpec((_R, sd), lambda i: (i, 0)),
        ] + _wspecs(ed, bd, sd),
        out_specs=pl.BlockSpec((_R, 2 * _HF), lambda i: (i, 0)),
        out_shape=jax.ShapeDtypeStruct((_NP, 2 * _HF), jnp.float32),
    )(c2, e_emb, b_emb, s_emb, *w)


def _tc_mid(c2, plo, phi, da, db, w, n, ed, bd, sd):
    def body(c_ref, plo_ref, phi_ref, da_ref, db_ref, ewt, eb1, bwt, bb1,
             swt, sb1, h_ref):
        sc = jnp.sqrt(c_ref[0, 0])
        e1, b1, s1 = _agg_split(plo_ref, phi_ref, da_ref, db_ref, sc)
        h_ref[...] = _fuse_pre(e1, b1, s1, ewt[...], eb1[...], bwt[...],
                               bb1[...], swt[...], sb1[...], sc)

    return pl.pallas_call(
        body,
        grid=(n // _R,),
        in_specs=[
            pl.BlockSpec(memory_space=pltpu.SMEM),
            pl.BlockSpec((_R, _HF), lambda i: (i, 0)),
            pl.BlockSpec((_R, _HF), lambda i: (i, 0)),
            pl.BlockSpec((_R, _DW), lambda i: (i, 0)),
            pl.BlockSpec((_R, _DW), lambda i: (i, 0)),
        ] + _wspecs(ed, bd, sd),
        out_specs=pl.BlockSpec((_R, 2 * _HF), lambda i: (i, 0)),
        out_shape=jax.ShapeDtypeStruct((_NP, 2 * _HF), jnp.float32),
    )(c2, plo, phi, da, db, *w)


def _tc_post(c2, plo, phi, da, db, n, ed, bd, sd):
    def body(c_ref, plo_ref, phi_ref, da_ref, db_ref, e_ref, b_ref, s_ref):
        sc = jnp.sqrt(c_ref[0, 0])
        e1, b1, s1 = _agg_split(plo_ref, phi_ref, da_ref, db_ref, sc)
        e_ref[...] = e1
        b_ref[...] = b1
        s_ref[...] = s1

    return pl.pallas_call(
        body,
        grid=(n // _R,),
        in_specs=[
            pl.BlockSpec(memory_space=pltpu.SMEM),
            pl.BlockSpec((_R, _HF), lambda i: (i, 0)),
            pl.BlockSpec((_R, _HF), lambda i: (i, 0)),
            pl.BlockSpec((_R, _DW), lambda i: (i, 0)),
            pl.BlockSpec((_R, _DW), lambda i: (i, 0)),
        ],
        out_specs=[
            pl.BlockSpec((_R, ed), lambda i: (i, 0)),
            pl.BlockSpec((_R, bd), lambda i: (i, 0)),
            pl.BlockSpec((_R, sd), lambda i: (i, 0)),
        ],
        out_shape=[
            jax.ShapeDtypeStruct((n, ed), jnp.float32),
            jax.ShapeDtypeStruct((n, bd), jnp.float32),
            jax.ShapeDtypeStruct((n, sd), jnp.float32),
        ],
    )(c2, plo, phi, da, db)


def _sc_agg(hview, srce3, srco3, dst3, z64, z16, ones_in, with_deg):
    n = hview.shape[0] // 2
    nch, cb = dst3.shape[1], dst3.shape[2]
    rpt = n // _NS

    mesh = plsc.VectorSubcoreMesh(core_axis_name="c", subcore_axis_name="s",
                                  num_cores=_NC, num_subcores=_NS)
    out_type = [jax.ShapeDtypeStruct((n, _HF), jnp.float32),
                jax.ShapeDtypeStruct((n, _HF), jnp.float32)]
    nr, ni = 5, 10
    scratch = [
        pltpu.VMEM((ni, cb), jnp.int32),
        pltpu.VMEM((ni, cb), jnp.int32),
        pltpu.VMEM((nr, cb, _HF), jnp.float32),
        pltpu.VMEM_SHARED((n, _HF), jnp.float32),
    ]
    scratch += [pltpu.SemaphoreType.DMA] * nr
    scratch += [pltpu.SemaphoreType.DMA] * ni
    if with_deg:
        out_type.append(jax.ShapeDtypeStruct((n, _DW), jnp.float32))
        out_type.append(jax.ShapeDtypeStruct((n, _DW), jnp.float32))
        scratch += [
            pltpu.VMEM((cb, _DW), jnp.float32),
            pltpu.VMEM_SHARED((n, _DW), jnp.float32),
        ]

    def body(hview_hbm, srce_hbm, srco_hbm, dst_hbm, z64_hbm, *rest):
        if with_deg:
            z16_hbm, ones_hbm = rest[:2]
            rest = rest[2:]
        plo_out, phi_out = rest[:2]
        rest = rest[2:]
        if with_deg:
            da_out, db_out = rest[:2]
            rest = rest[2:]
        sidx, didx, rows, acc = rest[:4]
        gsems = rest[4:4 + nr]
        isems = rest[4 + nr:4 + nr + ni]
        if with_deg:
            ones, dacc = rest[4 + nr + ni:]
        cid = lax.axis_index("c")
        sid = lax.axis_index("s")
        base = sid * rpt

        pltpu.sync_copy(z64_hbm, acc.at[pl.ds(base, rpt)])
        if with_deg:
            pltpu.sync_copy(z16_hbm, dacc.at[pl.ds(base, rpt)])
            pltpu.sync_copy(ones_hbm, ones)
        plsc.subcore_barrier()

        def _idx_issue(j, m):
            @pl.when(cid == 0)
            def _():
                pltpu.async_copy(srce_hbm.at[sid, j], sidx.at[m], isems[m])

            @pl.when(cid == 1)
            def _():
                pltpu.async_copy(srco_hbm.at[sid, j], sidx.at[m], isems[m])
            pltpu.async_copy(dst_hbm.at[sid, j], didx.at[m], isems[m])

        def _idx_wait(j, m):
            pltpu.make_async_copy(dst_hbm.at[sid, j], sidx.at[m],
                                  isems[m]).wait()
            pltpu.make_async_copy(dst_hbm.at[sid, j], didx.at[m],
                                  isems[m]).wait()

        def _gather(j, b, m):
            pltpu.async_copy(hview_hbm.at[sidx.at[m]], rows.at[b], gsems[b])

        def _gwait(j, b, m):
            pltpu.make_async_copy(hview_hbm.at[sidx.at[m]], rows.at[b],
                                  gsems[b]).wait()

        def _scat(j, b, m, par):
            pltpu.sync_copy(rows.at[b], acc.at[didx.at[m]], add=True)
            if with_deg:
                @pl.when(cid == par)
                def _():
                    pltpu.sync_copy(ones, dacc.at[didx.at[m]], add=True)

        for j in range(ni - 1):
            _idx_issue(j, j % ni)
        for j in range(nr - 1):
            _idx_wait(j, j % ni)
            _gather(j, j % nr, j % ni)

        def _slot(g, carry):
            for t in range(ni):
                j = ni * g + t
                b = t % nr
                _gwait(j, b, t)
                _scat(j, b, t, t % 2)

                @pl.when(j + ni - 1 < nch)
                def _():
                    _idx_issue(j + ni - 1, (t + ni - 1) % ni)

                @pl.when(j + nr - 1 < nch)
                def _():
                    _idx_wait(j + nr - 1, (t + nr - 1) % ni)
                    _gather(j + nr - 1, (b + nr - 1) % nr, (t + nr - 1) % ni)
            return carry
        lax.fori_loop(0, nch // ni, _slot, 0)
        plsc.subcore_barrier()

        sl = pl.ds(base, rpt)

        @pl.when(cid == 0)
        def _():
            pltpu.sync_copy(acc.at[sl], plo_out.at[sl])

        @pl.when(cid == 1)
        def _():
            pltpu.sync_copy(acc.at[sl], phi_out.at[sl])
        if with_deg:
            @pl.when(cid == 0)
            def _():
                pltpu.sync_copy(dacc.at[sl], da_out.at[sl])

            @pl.when(cid == 1)
            def _():
                pltpu.sync_copy(dacc.at[sl], db_out.at[sl])

    f = pl.kernel(body, out_type=out_type, mesh=mesh, scratch_types=scratch,
                  compiler_params=pltpu.CompilerParams(use_tc_tiling_on_sc=False))
    if with_deg:
        return f(hview, srce3, srco3, dst3, z64, z16, ones_in)
    return f(hview, srce3, srco3, dst3, z64)


def kernel(e_emb, b_emb, s_emb, edge_index, b_curvature, s_curvature,
           eW, eb, bW, bb, sW, sb):
    n, ed = e_emb.shape
    bd = b_emb.shape[1]
    sd = s_emb.shape[1]
    e_edges = edge_index.shape[1]
    per_t = e_edges // _NS
    cb = 125
    nch = per_t // cb
    src = edge_index[0]
    srce3 = (src * 2).reshape(_NS, nch, cb)
    srco3 = (src * 2 + 1).reshape(_NS, nch, cb)
    dst3 = edge_index[1].reshape(_NS, nch, cb)
    c2 = b_curvature.reshape(1, 1)

    def wlayer(l):
        return (eW[l].T, eb[l].reshape(1, ed), bW[l].T, bb[l].reshape(1, bd),
                sW[l].T, sb[l].reshape(1, sd))

    rpt = _NP // _NS
    z64 = jnp.zeros((rpt, _HF), jnp.float32)
    z16 = jnp.zeros((rpt, _DW), jnp.float32)
    ones_in = jnp.ones((cb, _DW), jnp.float32)

    h0 = _tc_pre(c2, e_emb, b_emb, s_emb, wlayer(0))
    plo0, phi0, d0a, d0b = _sc_agg(h0.reshape(2 * _NP, _HF), srce3, srco3,
                                   dst3, z64, z16, ones_in, with_deg=True)
    h1 = _tc_mid(c2, plo0, phi0, d0a, d0b, wlayer(1), n, ed, bd, sd)
    plo1, phi1 = _sc_agg(h1.reshape(2 * _NP, _HF), srce3, srco3, dst3,
                         z64, z16, ones_in, with_deg=False)
    e2, b2, s2 = _tc_post(c2, plo1, phi1, d0a, d0b, n, ed, bd, sd)
    return (e2, b2, s2)
